# Initial kernel scaffold; baseline (speedup 1.0000x reference)
import jax
import jax.numpy as jnp
from jax.experimental import pallas as pl


def kernel(belief, probs, bernoulli_uniforms, edge_index):
    raise NotImplementedError("write your pallas kernel here")


# speedup vs baseline: 39.3610x; 39.3610x over previous
import functools

import jax
import jax.numpy as jnp
from jax import lax
from jax.experimental import pallas as pl
from jax.experimental.pallas import tpu as pltpu
from jax.experimental.pallas import tpu_sc as plsc

N = 100000
E = 6400000
TRIALS = 10

NC = 2
NS = 16
NW = NC * NS
L = 16

N_PAD = 100352
SLICE = N_PAD // NS
B = 128
ROWS = 50048
ROWS_W = ROWS // NW
K = 4
CHUNKS = ROWS_W // K


def _payoff_body(belief_ref, probs_ref, u_ref, out_ref):
    mask = belief_ref[...] > 0.5
    p = probs_ref[...]
    cnt = jnp.zeros(belief_ref.shape, jnp.int32)
    for t in range(TRIALS):
        cnt += (u_ref[t] < p).astype(jnp.int32)
    word = jnp.where(mask, cnt + (TRIALS << 16), 0)
    out_ref[...] = word


def _payoff_words(belief2d, probs2d, u3d):
    nblk = N_PAD // (8 * 128)
    return pl.pallas_call(
        _payoff_body,
        grid=(nblk,),
        in_specs=[
            pl.BlockSpec((8, 128), lambda i: (i, 0)),
            pl.BlockSpec((8, 128), lambda i: (i, 0)),
            pl.BlockSpec((TRIALS, 8, 128), lambda i: (0, i, 0)),
        ],
        out_specs=pl.BlockSpec((8, 128), lambda i: (i, 0)),
        out_shape=jax.ShapeDtypeStruct((N_PAD // 128, 128), jnp.int32),
    )(belief2d, probs2d, u3d)


def _merge_body(p_ref, t_ref, po_ref, to_ref):
    po_ref[...] = p_ref[0] + p_ref[1]
    to_ref[...] = t_ref[0] + t_ref[1]


def _merge(p_part, t_part):
    return pl.pallas_call(
        _merge_body,
        out_shape=[
            jax.ShapeDtypeStruct((N_PAD // 128, 128), jnp.float32),
            jax.ShapeDtypeStruct((N_PAD // 128, 128), jnp.float32),
        ],
    )(p_part, t_part)


def _edge_kernel(words_hbm, src_hbm, dst_hbm,
                 p_out, t_out,
                 table, acc_p, acc_t,
                 src_buf, dst_buf, wrow, lo_buf, hi_buf, zbuf):
    c = lax.axis_index("c")
    s = lax.axis_index("s")
    w = s * NC + c

    node_base = s * SLICE
    pltpu.sync_copy(words_hbm.at[pl.ds(node_base, SLICE)],
                    table.at[pl.ds(node_base, SLICE)])

    def _zfill(i, _):
        zbuf[pl.ds(i * L, L)] = jnp.zeros((L,), jnp.float32)
        return _
    lax.fori_loop(0, SLICE // L, _zfill, None)
    pltpu.sync_copy(zbuf, acc_p.at[pl.ds(node_base, SLICE)])
    pltpu.sync_copy(zbuf, acc_t.at[pl.ds(node_base, SLICE)])
    plsc.subcore_barrier()

    row_base = w * ROWS_W

    def _chunk(g, _):
        r0 = row_base + g * K
        pltpu.sync_copy(src_hbm.at[pl.ds(r0, K)], src_buf)
        pltpu.sync_copy(dst_hbm.at[pl.ds(r0, K)], dst_buf)
        for j in range(K):
            pltpu.sync_copy(table.at[src_buf.at[j]], wrow)
            for i in range(B // L):
                w16 = wrow[pl.ds(i * L, L)]
                lo_buf[pl.ds(i * L, L)] = (w16 & 0xFFFF).astype(jnp.float32)
                hi_buf[pl.ds(i * L, L)] = (w16 >> 16).astype(jnp.float32)
            pltpu.sync_copy(lo_buf, acc_p.at[dst_buf.at[j]], add=True)
            pltpu.sync_copy(hi_buf, acc_t.at[dst_buf.at[j]], add=True)
        return _
    lax.fori_loop(0, CHUNKS, _chunk, None)

    plsc.subcore_barrier()
    pltpu.sync_copy(acc_p.at[pl.ds(node_base, SLICE)],
                    p_out.at[c, pl.ds(node_base, SLICE)])
    pltpu.sync_copy(acc_t.at[pl.ds(node_base, SLICE)],
                    t_out.at[c, pl.ds(node_base, SLICE)])


_edge_call = pl.kernel(
    _edge_kernel,
    out_type=[
        jax.ShapeDtypeStruct((NC, N_PAD), jnp.float32),
        jax.ShapeDtypeStruct((NC, N_PAD), jnp.float32),
    ],
    mesh=plsc.VectorSubcoreMesh(core_axis_name="c", subcore_axis_name="s"),
    scratch_types=[
        pltpu.MemorySpace.VMEM_SHARED((N_PAD,), jnp.int32),
        pltpu.MemorySpace.VMEM_SHARED((N_PAD,), jnp.float32),
        pltpu.MemorySpace.VMEM_SHARED((N_PAD,), jnp.float32),
        pltpu.VMEM((K, B), jnp.int32),
        pltpu.VMEM((K, B), jnp.int32),
        pltpu.VMEM((B,), jnp.int32),
        pltpu.VMEM((B,), jnp.float32),
        pltpu.VMEM((B,), jnp.float32),
        pltpu.VMEM((SLICE,), jnp.float32),
    ],
)


def kernel(belief, probs, bernoulli_uniforms, edge_index):
    pad_n = N_PAD - N
    belief2d = jnp.pad(belief, (0, pad_n)).reshape(N_PAD // 128, 128)
    probs2d = jnp.pad(probs, (0, pad_n)).reshape(N_PAD // 128, 128)
    u3d = jnp.pad(bernoulli_uniforms.T, ((0, 0), (0, pad_n))).reshape(
        TRIALS, N_PAD // 128, 128)

    words = _payoff_words(belief2d, probs2d, u3d).reshape(N_PAD)

    pad_e = ROWS * B - E
    src_p = jnp.concatenate(
        [edge_index[0], jnp.zeros((pad_e,), jnp.int32)]).reshape(ROWS, B)
    trash = N + (jnp.arange(pad_e, dtype=jnp.int32) % pad_n)
    dst_p = jnp.concatenate([edge_index[1], trash]).reshape(ROWS, B)

    p_part, t_part = _edge_call(words, src_p, dst_p)
    p_sum, t_sum = _merge(p_part.reshape(NC, N_PAD // 128, 128),
                          t_part.reshape(NC, N_PAD // 128, 128))
    return jnp.stack([p_sum.reshape(N_PAD)[:N],
                      t_sum.reshape(N_PAD)[:N]], axis=1)


# speedup vs baseline: 103.2178x; 2.6223x over previous
import jax
import jax.numpy as jnp
from jax import lax
from jax.experimental import pallas as pl
from jax.experimental.pallas import tpu as pltpu
from jax.experimental.pallas import tpu_sc as plsc

N = 100000
E = 6400000
TRIALS = 10

NC = 2
NS = 16
NW = NC * NS
L = 16

N_PAD = 100352
SLICE = N_PAD // NS
B = 128
ROWS = 50176
ROWS_W = ROWS // NW
K = 4
CHUNKS = ROWS_W // K


def _payoff_body(belief_ref, probs_ref, u_ref, out_ref):
    mask = belief_ref[...] > 0.5
    p = probs_ref[...]
    cnt = jnp.zeros(belief_ref.shape, jnp.int32)
    for t in range(TRIALS):
        cnt += (u_ref[t] < p).astype(jnp.int32)
    out_ref[...] = jnp.where(mask, cnt + (TRIALS << 16), 0)


def _payoff_words(belief2d, probs2d, u3d):
    nblk = N_PAD // (8 * 128)
    return pl.pallas_call(
        _payoff_body,
        grid=(nblk,),
        in_specs=[
            pl.BlockSpec((8, 128), lambda i: (i, 0)),
            pl.BlockSpec((8, 128), lambda i: (i, 0)),
            pl.BlockSpec((TRIALS, 8, 128), lambda i: (0, i, 0)),
        ],
        out_specs=pl.BlockSpec((8, 128), lambda i: (i, 0)),
        out_shape=jax.ShapeDtypeStruct((N_PAD // 128, 128), jnp.int32),
    )(belief2d, probs2d, u3d)


def _merge_body(part_ref, p_ref, t_ref):
    p_ref[...] = part_ref[0] + part_ref[2]
    t_ref[...] = part_ref[1] + part_ref[3]


def _merge(parts):
    return pl.pallas_call(
        _merge_body,
        out_shape=[
            jax.ShapeDtypeStruct((N_PAD // 128, 128), jnp.float32),
            jax.ShapeDtypeStruct((N_PAD // 128, 128), jnp.float32),
        ],
    )(parts)


def _edge_kernel(words_hbm, src_hbm, dst_hbm, zeros_hbm,
                 out,
                 table, acc_p, acc_t,
                 src_buf, dst_buf, wbuf, lo_buf, hi_buf,
                 se0, se1, se2, se3, sw0, sw1, ss0, ss1):
    c = lax.axis_index("c")
    s = lax.axis_index("s")
    w = s * NC + c
    se = (se0, se1, se2, se3)
    sw = (sw0, sw1)
    ss = (ss0, ss1)

    node_base = s * SLICE
    pltpu.sync_copy(words_hbm.at[pl.ds(node_base, SLICE)],
                    table.at[pl.ds(node_base, SLICE)])
    pltpu.sync_copy(zeros_hbm.at[pl.ds(node_base, SLICE)],
                    acc_p.at[pl.ds(node_base, SLICE)])
    pltpu.sync_copy(zeros_hbm.at[pl.ds(node_base, SLICE)],
                    acc_t.at[pl.ds(node_base, SLICE)])
    plsc.subcore_barrier()

    row_base = w * ROWS_W

    def _load(chunk, slot):
        r0 = row_base + chunk * K
        pltpu.async_copy(src_hbm.at[pl.ds(r0, K)], src_buf.at[slot], se[slot])
        pltpu.async_copy(dst_hbm.at[pl.ds(r0, K)], dst_buf.at[slot], se[slot])

    def _wait_load(slot):
        pltpu.make_async_copy(src_hbm.at[pl.ds(0, K)], src_buf.at[slot],
                              se[slot]).wait()
        pltpu.make_async_copy(dst_hbm.at[pl.ds(0, K)], dst_buf.at[slot],
                              se[slot]).wait()

    def _drain_scatter(p):
        for j in range(K):
            pltpu.make_async_copy(lo_buf.at[p, j],
                                  acc_p.at[dst_buf.at[p, j]], ss[p]).wait()
            pltpu.make_async_copy(hi_buf.at[p, j],
                                  acc_t.at[dst_buf.at[p, j]], ss[p]).wait()

    _load(0, 0)
    _load(1, 1)

    def _g4(g4, carry):
        for u in range(4):
            p = u & 1
            G = g4 * 4 + u
            _wait_load(u)

            @pl.when(G >= 2)
            def _():
                _drain_scatter(p)

            @pl.when(G + 2 < CHUNKS)
            def _():
                _load(G + 2, (u + 2) % 4)

            gds = [pltpu.async_copy(table.at[src_buf.at[u, j]],
                                    wbuf.at[p, j], sw[p]) for j in range(K)]
            for d in gds:
                d.wait()

            for j in range(K):
                for i in range(B // L):
                    w16 = wbuf[p, j, pl.ds(i * L, L)]
                    lo_buf[p, j, pl.ds(i * L, L)] = (
                        w16 & 0xFFFF).astype(jnp.float32)
                    hi_buf[p, j, pl.ds(i * L, L)] = (
                        w16 >> 16).astype(jnp.float32)

            for j in range(K):
                pltpu.async_copy(lo_buf.at[p, j], acc_p.at[dst_buf.at[u, j]],
                                 ss[p], add=True)
                pltpu.async_copy(hi_buf.at[p, j], acc_t.at[dst_buf.at[u, j]],
                                 ss[p], add=True)
        return carry
    lax.fori_loop(0, CHUNKS // 4, _g4, None)

    for p in range(2):
        _drain_scatter(p)

    plsc.subcore_barrier()
    pltpu.sync_copy(acc_p.at[pl.ds(node_base, SLICE)],
                    out.at[c, 0, pl.ds(node_base, SLICE)])
    pltpu.sync_copy(acc_t.at[pl.ds(node_base, SLICE)],
                    out.at[c, 1, pl.ds(node_base, SLICE)])


_edge_call = pl.kernel(
    _edge_kernel,
    out_type=jax.ShapeDtypeStruct((NC, 2, N_PAD), jnp.float32),
    mesh=plsc.VectorSubcoreMesh(core_axis_name="c", subcore_axis_name="s"),
    scratch_types=[
        pltpu.MemorySpace.VMEM_SHARED((N_PAD,), jnp.int32),
        pltpu.MemorySpace.VMEM_SHARED((N_PAD,), jnp.float32),
        pltpu.MemorySpace.VMEM_SHARED((N_PAD,), jnp.float32),
        pltpu.VMEM((4, K, B), jnp.int32),
        pltpu.VMEM((4, K, B), jnp.int32),
        pltpu.VMEM((2, K, B), jnp.int32),
        pltpu.VMEM((2, K, B), jnp.float32),
        pltpu.VMEM((2, K, B), jnp.float32),
    ] + [pltpu.SemaphoreType.DMA] * 8,
)


def kernel(belief, probs, bernoulli_uniforms, edge_index):
    pad_n = N_PAD - N
    belief2d = jnp.pad(belief, (0, pad_n)).reshape(N_PAD // 128, 128)
    probs2d = jnp.pad(probs, (0, pad_n)).reshape(N_PAD // 128, 128)
    u3d = jnp.pad(bernoulli_uniforms.T, ((0, 0), (0, pad_n))).reshape(
        TRIALS, N_PAD // 128, 128)

    words = _payoff_words(belief2d, probs2d, u3d).reshape(N_PAD)

    pad_e = ROWS * B - E
    src_p = jnp.concatenate(
        [edge_index[0], jnp.zeros((pad_e,), jnp.int32)]).reshape(ROWS, B)
    trash = N + (jnp.arange(pad_e, dtype=jnp.int32) % pad_n)
    dst_p = jnp.concatenate([edge_index[1], trash]).reshape(ROWS, B)

    zeros = jnp.zeros((N_PAD,), jnp.float32)
    parts = _edge_call(words, src_p, dst_p, zeros)
    p_sum, t_sum = _merge(parts.reshape(NC * 2, N_PAD // 128, 128))
    return jnp.stack([p_sum.reshape(N_PAD)[:N],
                      t_sum.reshape(N_PAD)[:N]], axis=1)


# speedup vs baseline: 107.7034x; 1.0435x over previous
import jax
import jax.numpy as jnp
from jax import lax
from jax.experimental import pallas as pl
from jax.experimental.pallas import tpu as pltpu
from jax.experimental.pallas import tpu_sc as plsc

N = 100000
E = 6400000
TRIALS = 10

NC = 2
NS = 16
NW = NC * NS
L = 16

N_PAD = 100352
SLICE = N_PAD // NS
B = 128
ROWS = E // B
K = 8
ROWS_HI = 1568
ROWS_LO = 1560


def _payoff_body(belief_ref, probs_ref, u_ref, out_ref):
    mask = belief_ref[...] > 0.5
    p = probs_ref[...]
    cnt = jnp.zeros(belief_ref.shape, jnp.int32)
    for t in range(TRIALS):
        cnt += (u_ref[t] < p).astype(jnp.int32)
    out_ref[...] = jnp.where(mask, cnt + (TRIALS << 16), 0)


def _payoff_words(belief2d, probs2d, u3d):
    nblk = N_PAD // (8 * 128)
    return pl.pallas_call(
        _payoff_body,
        grid=(nblk,),
        in_specs=[
            pl.BlockSpec((8, 128), lambda i: (i, 0)),
            pl.BlockSpec((8, 128), lambda i: (i, 0)),
            pl.BlockSpec((TRIALS, 8, 128), lambda i: (0, i, 0)),
        ],
        out_specs=pl.BlockSpec((8, 128), lambda i: (i, 0)),
        out_shape=jax.ShapeDtypeStruct((N_PAD // 128, 128), jnp.int32),
    )(belief2d, probs2d, u3d)


def _merge_body(part_ref, p_ref, t_ref):
    p_ref[...] = part_ref[0] + part_ref[2]
    t_ref[...] = part_ref[1] + part_ref[3]


def _merge(parts):
    return pl.pallas_call(
        _merge_body,
        out_shape=[
            jax.ShapeDtypeStruct((N_PAD // 128, 128), jnp.float32),
            jax.ShapeDtypeStruct((N_PAD // 128, 128), jnp.float32),
        ],
    )(parts)


def _edge_kernel(words_hbm, edges_hbm, zeros_hbm,
                 out,
                 table, acc_p, acc_t,
                 src_buf, dst_buf, wbuf, lo_buf, hi_buf,
                 se0, se1, se2, se3, sw0, sw1, ss0, ss1):
    c = lax.axis_index("c")
    s = lax.axis_index("s")
    w = s * NC + c
    se = (se0, se1, se2, se3)
    sw = (sw0, sw1)
    ss = (ss0, ss1)

    node_base = s * SLICE
    pltpu.sync_copy(words_hbm.at[pl.ds(node_base, SLICE)],
                    table.at[pl.ds(node_base, SLICE)])
    pltpu.sync_copy(zeros_hbm.at[pl.ds(node_base, SLICE)],
                    acc_p.at[pl.ds(node_base, SLICE)])
    pltpu.sync_copy(zeros_hbm.at[pl.ds(node_base, SLICE)],
                    acc_t.at[pl.ds(node_base, SLICE)])
    plsc.subcore_barrier()

    row_base = w * ROWS_LO + 8 * jnp.minimum(w, 10)
    n_chunks = (ROWS_LO // K) + jnp.where(w < 10, 1, 0)

    def _load(chunk, slot):
        r0 = row_base + chunk * K
        pltpu.async_copy(edges_hbm.at[0, pl.ds(r0, K)], src_buf.at[slot],
                         se[slot])
        pltpu.async_copy(edges_hbm.at[1, pl.ds(r0, K)], dst_buf.at[slot],
                         se[slot])

    def _wait_load(slot):
        pltpu.make_async_copy(edges_hbm.at[0, pl.ds(0, K)], src_buf.at[slot],
                              se[slot]).wait()
        pltpu.make_async_copy(edges_hbm.at[1, pl.ds(0, K)], dst_buf.at[slot],
                              se[slot]).wait()

    def _drain_scatter(p):
        for j in range(K):
            pltpu.make_async_copy(lo_buf.at[p, j],
                                  acc_p.at[dst_buf.at[p, j]], ss[p]).wait()
            pltpu.make_async_copy(hi_buf.at[p, j],
                                  acc_t.at[dst_buf.at[p, j]], ss[p]).wait()

    def _decode(p, j):
        for i in range(B // L):
            w16 = wbuf[p, j, pl.ds(i * L, L)]
            lo_buf[p, j, pl.ds(i * L, L)] = (w16 & 0xFFFF).astype(jnp.float32)
            hi_buf[p, j, pl.ds(i * L, L)] = (w16 >> 16).astype(jnp.float32)

    def _subiter(G, u):
        p = u & 1
        _wait_load(u)

        @pl.when(G >= 2)
        def _():
            _drain_scatter(p)

        @pl.when(G + 2 < n_chunks)
        def _():
            _load(G + 2, (u + 2) % 4)

        gds = [pltpu.async_copy(table.at[src_buf.at[u, j]],
                                wbuf.at[p, j], sw[p]) for j in range(K)]
        for d in gds:
            d.wait()

        for j in range(K):
            _decode(p, j)

        for j in range(K):
            pltpu.async_copy(lo_buf.at[p, j], acc_p.at[dst_buf.at[u, j]],
                             ss[p], add=True)
            pltpu.async_copy(hi_buf.at[p, j], acc_t.at[dst_buf.at[u, j]],
                             ss[p], add=True)

    _load(0, 0)
    _load(1, 1)

    nfull = n_chunks // 4

    def _g4(g4, carry):
        for u in range(4):
            _subiter(g4 * 4 + u, u)
        return carry
    lax.fori_loop(0, nfull, _g4, None)

    @pl.when(n_chunks % 4 != 0)
    def _():
        for u in range(3):
            _subiter(nfull * 4 + u, u)

    for p in range(2):
        _drain_scatter(p)

    plsc.subcore_barrier()
    pltpu.sync_copy(acc_p.at[pl.ds(node_base, SLICE)],
                    out.at[c, 0, pl.ds(node_base, SLICE)])
    pltpu.sync_copy(acc_t.at[pl.ds(node_base, SLICE)],
                    out.at[c, 1, pl.ds(node_base, SLICE)])


_edge_call = pl.kernel(
    _edge_kernel,
    out_type=jax.ShapeDtypeStruct((NC, 2, N_PAD), jnp.float32),
    mesh=plsc.VectorSubcoreMesh(core_axis_name="c", subcore_axis_name="s"),
    scratch_types=[
        pltpu.MemorySpace.VMEM_SHARED((N_PAD,), jnp.int32),
        pltpu.MemorySpace.VMEM_SHARED((N_PAD,), jnp.float32),
        pltpu.MemorySpace.VMEM_SHARED((N_PAD,), jnp.float32),
        pltpu.VMEM((4, K, B), jnp.int32),
        pltpu.VMEM((4, K, B), jnp.int32),
        pltpu.VMEM((2, K, B), jnp.int32),
        pltpu.VMEM((2, K, B), jnp.float32),
        pltpu.VMEM((2, K, B), jnp.float32),
    ] + [pltpu.SemaphoreType.DMA] * 8,
)


def kernel(belief, probs, bernoulli_uniforms, edge_index):
    pad_n = N_PAD - N
    belief2d = jnp.pad(belief, (0, pad_n)).reshape(N_PAD // 128, 128)
    probs2d = jnp.pad(probs, (0, pad_n)).reshape(N_PAD // 128, 128)
    u3d = jnp.pad(bernoulli_uniforms.T, ((0, 0), (0, pad_n))).reshape(
        TRIALS, N_PAD // 128, 128)

    words = _payoff_words(belief2d, probs2d, u3d).reshape(N_PAD)

    edges3 = edge_index.reshape(2, ROWS, B)
    zeros = jnp.zeros((N_PAD,), jnp.float32)
    parts = _edge_call(words, edges3, zeros)
    p_sum, t_sum = _merge(parts.reshape(NC * 2, N_PAD // 128, 128))
    return jnp.stack([p_sum.reshape(N_PAD)[:N],
                      t_sum.reshape(N_PAD)[:N]], axis=1)
